# baseline (speedup 1.0000x reference)
import functools
import math

import jax
import jax.numpy as jnp
from jax import lax
from jax.experimental import pallas as pl
from jax.experimental.pallas import tpu as pltpu
from jax.experimental.pallas import tpu_sc as plsc

_VOCAB = 32000
_N = 4096
_FILL = 0.1 / (_VOCAB - 2)
_CONF = 1.0 - 0.1
_C_ROW = 0.1 * math.log(_FILL) + _CONF * math.log(_CONF)

_NC = 2
_NS = 16
_NW = _NC * _NS
_RPW = _N // _NW
_VPW = _RPW // 16


def _sc_body(pred_hbm, tgt_hbm, out_hbm, t_v, idx_t, idx_0,
             vals_t, vals_0, obuf, sem1, sem2):
    wid = lax.axis_index("s") * _NC + lax.axis_index("c")
    base = wid * _RPW
    pltpu.sync_copy(tgt_hbm.at[pl.ds(base, _RPW)], t_v)
    for j in range(_VPW):
        t = t_v[pl.ds(j * 16, 16)]
        rows = base + j * 16 + lax.broadcasted_iota(jnp.int32, (16,), 0)
        r0 = rows * _VOCAB
        idx_t[pl.ds(j * 16, 16)] = r0 + t
        idx_0[pl.ds(j * 16, 16)] = r0
    c1 = pltpu.async_copy(pred_hbm.at[idx_t], vals_t, sem1)
    c2 = pltpu.async_copy(pred_hbm.at[idx_0], vals_0, sem2)
    c1.wait()
    c2.wait()
    accp = jnp.zeros((16,), jnp.float32)
    acc0 = jnp.zeros((16,), jnp.float32)
    accn = jnp.zeros((16,), jnp.float32)
    for j in range(_VPW):
        m = t_v[pl.ds(j * 16, 16)] != 0
        accp = accp + jnp.where(m, vals_t[pl.ds(j * 16, 16)], 0.0)
        acc0 = acc0 + jnp.where(m, vals_0[pl.ds(j * 16, 16)], 0.0)
        accn = accn + jnp.where(m, 1.0, 0.0)
    obuf[pl.ds(0, 16)] = accp
    obuf[pl.ds(16, 16)] = acc0
    obuf[pl.ds(32, 16)] = accn
    obuf[pl.ds(48, 16)] = jnp.zeros((16,), jnp.float32)
    pltpu.sync_copy(obuf, out_hbm.at[wid])


_sc_gather = functools.partial(
    pl.kernel,
    mesh=plsc.VectorSubcoreMesh(core_axis_name="c", subcore_axis_name="s"),
    out_type=jax.ShapeDtypeStruct((_NW, 64), jnp.float32),
    scratch_types=[
        pltpu.VMEM((_RPW,), jnp.int32),
        pltpu.VMEM((_RPW,), jnp.int32),
        pltpu.VMEM((_RPW,), jnp.int32),
        pltpu.VMEM((_RPW,), jnp.float32),
        pltpu.VMEM((_RPW,), jnp.float32),
        pltpu.VMEM((64,), jnp.float32),
        pltpu.SemaphoreType.DMA,
        pltpu.SemaphoreType.DMA,
    ],
)(_sc_body)

_RB = 512
_BV = 3200
_GR = _N // _RB
_GV = _VOCAB // _BV


def _tc_body(t_ref, p_ref, x_ref, out_ref, acc_ref):
    i = pl.program_id(0)
    j = pl.program_id(1)

    @pl.when((i == 0) & (j == 0))
    def _init():
        acc_ref[0] = 0.0

    x = x_ref[...]
    tcol = t_ref[:, 0:1]
    valid = tcol != 0

    srows = jnp.sum(x, axis=1, keepdims=True)
    acc_ref[0] += jnp.sum(jnp.where(valid, srows, 0.0))

    @pl.when((i == _GR - 1) & (j == _GV - 1))
    def _fin():
        p = p_ref[...]
        ptv = jnp.sum(p[:, 0:16])
        p0v = jnp.sum(p[:, 16:32])
        nv = jnp.sum(p[:, 32:48])
        out_ref[0, 0] = (nv * _C_ROW - _FILL * acc_ref[0]
                         + _FILL * p0v + (_FILL - _CONF) * ptv)


def kernel(predictions, targets):
    n = predictions.shape[0]
    flat = jnp.reshape(predictions, (-1,))
    partials = _sc_gather(flat, targets.astype(jnp.int32))
    t2 = jnp.broadcast_to(targets[:, None].astype(jnp.int32), (n, 128))
    out = pl.pallas_call(
        _tc_body,
        grid=(_GR, _GV),
        in_specs=[
            pl.BlockSpec((_RB, 128), lambda i, j: (i, 0)),
            pl.BlockSpec((_NW, 64), lambda i, j: (0, 0)),
            pl.BlockSpec((_RB, _BV), lambda i, j: (i, j)),
        ],
        out_specs=pl.BlockSpec((1, 1), lambda i, j: (0, 0),
                               memory_space=pltpu.SMEM),
        out_shape=jax.ShapeDtypeStruct((1, 1), jnp.float32),
        scratch_shapes=[pltpu.SMEM((1,), jnp.float32)],
        compiler_params=pltpu.CompilerParams(
            dimension_semantics=("arbitrary", "arbitrary")),
    )(t2, partials, predictions)
    return out[0, 0]


# speedup vs baseline: 2.9912x; 2.9912x over previous
import math

import jax
import jax.numpy as jnp
from jax.experimental import pallas as pl
from jax.experimental.pallas import tpu as pltpu

_VOCAB = 32000
_N = 4096
_FILL = 0.1 / (_VOCAB - 2)
_CONF = 1.0 - 0.1
_C_ROW = 0.1 * math.log(_FILL) + _CONF * math.log(_CONF)
_KMUL = _CONF / _FILL

_RB = 512
_BV = 3200
_GR = _N // _RB
_GV = _VOCAB // _BV


def _body(t_ref, x_ref, out_ref, acc_ref):
    i = pl.program_id(0)
    j = pl.program_id(1)

    @pl.when((i == 0) & (j == 0))
    def _init():
        acc_ref[0] = 0.0
        acc_ref[1] = 0.0
        acc_ref[2] = 0.0
        acc_ref[3] = 0.0

    x = x_ref[...]
    tcol = t_ref[:, 0:1]
    valid = tcol != 0

    lane = jax.lax.broadcasted_iota(jnp.int32, (_RB, _BV), 1)
    rel = tcol - j * _BV
    y = jnp.where(lane == rel, x * _KMUL, x)
    srows = jnp.sum(y, axis=1, keepdims=True)
    acc_ref[0] += jnp.sum(jnp.where(valid, srows, 0.0))

    @pl.when(j == 0)
    def _col0():
        acc_ref[2] += jnp.sum(jnp.where(valid, x[:, 0:1], 0.0))
        acc_ref[3] += jnp.sum(jnp.where(valid, 1.0, 0.0))

    @pl.when((i == _GR - 1) & (j == _GV - 1))
    def _fin():
        out_ref[0, 0] = (acc_ref[3] * _C_ROW - _FILL * acc_ref[0]
                         + _FILL * acc_ref[2])


def kernel(predictions, targets):
    n = predictions.shape[0]
    t2 = jnp.broadcast_to(targets[:, None].astype(jnp.int32), (n, 128))
    out = pl.pallas_call(
        _body,
        grid=(_GR, _GV),
        in_specs=[
            pl.BlockSpec((_RB, 128), lambda i, j: (i, 0)),
            pl.BlockSpec((_RB, _BV), lambda i, j: (i, j)),
        ],
        out_specs=pl.BlockSpec((1, 1), lambda i, j: (0, 0),
                               memory_space=pltpu.SMEM),
        out_shape=jax.ShapeDtypeStruct((1, 1), jnp.float32),
        scratch_shapes=[pltpu.SMEM((4,), jnp.float32)],
        compiler_params=pltpu.CompilerParams(
            dimension_semantics=("arbitrary", "arbitrary")),
    )(t2, predictions)
    return out[0, 0]
